# Initial kernel scaffold; baseline (speedup 1.0000x reference)
import jax
import jax.numpy as jnp
from jax.experimental import pallas as pl


def kernel(x, hour_table, day_table, W, b, pe, timestamps):
    raise NotImplementedError("write your pallas kernel here")


# speedup vs baseline: 3.9307x; 3.9307x over previous
import functools

import jax
import jax.numpy as jnp
from jax.experimental import pallas as pl

S_BLK = 512


def _pe_kernel(ts_ref, x_ref, pe_ref, ht_ref, dt_ref, w_ref, b_ref, o_ref):
    ts = ts_ref[...]
    hour = (ts // 3600) % 24
    day = (ts // 86400 + 3) % 7

    hw = jnp.dot(ht_ref[...], w_ref[0:256, :], preferred_element_type=jnp.float32)
    dw = jnp.dot(dt_ref[...], w_ref[256:512, :], preferred_element_type=jnp.float32)

    iota_h = jax.lax.broadcasted_iota(jnp.int32, (1, 24), 1)
    iota_d = jax.lax.broadcasted_iota(jnp.int32, (1, 7), 1)

    bias = b_ref[...]
    nb = ts.shape[1]
    for bb in range(nb):
        oh_h = (hour[:, bb:bb + 1] == iota_h).astype(jnp.float32)
        oh_d = (day[:, bb:bb + 1] == iota_d).astype(jnp.float32)
        temporal = (
            jnp.dot(oh_h, hw, preferred_element_type=jnp.float32)
            + jnp.dot(oh_d, dw, preferred_element_type=jnp.float32)
        )
        o_ref[:, bb, :] = x_ref[:, bb, :] + pe_ref[:, 0, :] + temporal + bias


def kernel(x, hour_table, day_table, W, b, pe, timestamps):
    S, B, D = x.shape
    ts_t = timestamps.T
    pe_s = pe[:S]
    b2 = b.reshape(1, D)

    grid = (S // S_BLK,)
    return pl.pallas_call(
        _pe_kernel,
        grid=grid,
        in_specs=[
            pl.BlockSpec((S_BLK, B), lambda i: (i, 0)),
            pl.BlockSpec((S_BLK, B, D), lambda i: (i, 0, 0)),
            pl.BlockSpec((S_BLK, 1, D), lambda i: (i, 0, 0)),
            pl.BlockSpec(hour_table.shape, lambda i: (0, 0)),
            pl.BlockSpec(day_table.shape, lambda i: (0, 0)),
            pl.BlockSpec(W.shape, lambda i: (0, 0)),
            pl.BlockSpec((1, D), lambda i: (0, 0)),
        ],
        out_specs=pl.BlockSpec((S_BLK, B, D), lambda i: (i, 0, 0)),
        out_shape=jax.ShapeDtypeStruct((S, B, D), jnp.float32),
    )(ts_t, x, pe_s, hour_table, day_table, W, b2)


# speedup vs baseline: 4.3570x; 1.1085x over previous
import functools

import jax
import jax.numpy as jnp
from jax.experimental import pallas as pl

S_BLK = 512


def _pe_kernel(ts_ref, x_ref, pe_ref, ht_ref, dt_ref, w_ref, b_ref, o_ref):
    ts = ts_ref[...]
    hour = (ts // 3600) % 24
    day = (ts // 86400 + 3) % 7

    hw = jnp.dot(ht_ref[...], w_ref[0:256, :], preferred_element_type=jnp.float32)
    hw = hw + b_ref[...]
    dw = jnp.dot(dt_ref[...], w_ref[256:512, :], preferred_element_type=jnp.float32)
    cat = jnp.concatenate([hw, dw], axis=0)

    nb = ts.shape[0]
    s_blk = ts.shape[1]
    iota_c = jax.lax.broadcasted_iota(jnp.int32, (31, s_blk), 0)
    dims = (((0,), (0,)), ((), ()))
    for bb in range(nb):
        hit = (hour[bb:bb + 1, :] == iota_c) | (day[bb:bb + 1, :] + 24 == iota_c)
        oh = hit.astype(jnp.float32)
        temporal = jax.lax.dot_general(
            oh, cat, dims, preferred_element_type=jnp.float32)
        o_ref[:, bb, :] = x_ref[:, bb, :] + pe_ref[:, 0, :] + temporal


def kernel(x, hour_table, day_table, W, b, pe, timestamps):
    S, B, D = x.shape
    pe_s = pe[:S]
    b2 = b.reshape(1, D)

    grid = (S // S_BLK,)
    return pl.pallas_call(
        _pe_kernel,
        grid=grid,
        in_specs=[
            pl.BlockSpec((B, S_BLK), lambda i: (0, i)),
            pl.BlockSpec((S_BLK, B, D), lambda i: (i, 0, 0)),
            pl.BlockSpec((S_BLK, 1, D), lambda i: (i, 0, 0)),
            pl.BlockSpec(hour_table.shape, lambda i: (0, 0)),
            pl.BlockSpec(day_table.shape, lambda i: (0, 0)),
            pl.BlockSpec(W.shape, lambda i: (0, 0)),
            pl.BlockSpec((1, D), lambda i: (0, 0)),
        ],
        out_specs=pl.BlockSpec((S_BLK, B, D), lambda i: (i, 0, 0)),
        out_shape=jax.ShapeDtypeStruct((S, B, D), jnp.float32),
    )(timestamps, x, pe_s, hour_table, day_table, W, b2)
